# baseline (speedup 1.0000x reference)
import functools

import jax
import jax.numpy as jnp
from jax import lax
from jax.experimental import pallas as pl
from jax.experimental.pallas import tpu as pltpu
from jax.experimental.pallas import tpu_sc as plsc

NC = 2
NS = 16
NW = NC * NS
LN = 128


def _mesh():
    return plsc.VectorSubcoreMesh(
        core_axis_name="c", subcore_axis_name="s", num_cores=NC, num_subcores=NS
    )


def _sc_degree(src_p, dst_p, r_acc):
    ch = src_p.shape[1]
    rpt = r_acc // NS

    def body(src_hbm, dst_hbm, degs_hbm, degd_hbm,
             src_v, dst_v, ones_v, zv, dsem, degs_sh, degd_sh):
        c = lax.axis_index("c")
        s = lax.axis_index("s")
        wid = s * NC + c
        pltpu.sync_copy(src_hbm.at[wid], src_v)
        pltpu.sync_copy(dst_hbm.at[wid], dst_v)
        for k in range(LN // 16):
            ones_v[pl.ds(k * 16, 16)] = jnp.ones((16,), jnp.float32)

        def _z(i, carry):
            zv[pl.ds(i * 16, 16)] = jnp.zeros((16,), jnp.float32)
            return carry
        lax.fori_loop(0, -(-rpt // 16), _z, 0)
        pltpu.sync_copy(zv.at[pl.ds(0, rpt)], degs_sh.at[pl.ds(s * rpt, rpt)])
        pltpu.sync_copy(zv.at[pl.ds(0, rpt)], degd_sh.at[pl.ds(s * rpt, rpt)])
        base = c * (NS * rpt) + s * rpt
        plsc.subcore_barrier()

        def _chunk(j, carry):
            pltpu.async_copy(ones_v, degs_sh.at[src_v.at[j]], dsem, add=True)
            pltpu.async_copy(ones_v, degd_sh.at[dst_v.at[j]], dsem, add=True)
            return carry
        lax.fori_loop(0, ch, _chunk, 0)

        def _drain(j, carry):
            pltpu.make_async_copy(ones_v, degs_sh.at[src_v.at[j]], dsem).wait()
            pltpu.make_async_copy(ones_v, degd_sh.at[dst_v.at[j]], dsem).wait()
            return carry
        lax.fori_loop(0, ch, _drain, 0)
        plsc.subcore_barrier()
        pltpu.sync_copy(degs_sh.at[pl.ds(s * rpt, rpt)],
                        degs_hbm.at[pl.ds(base, rpt)])
        pltpu.sync_copy(degd_sh.at[pl.ds(s * rpt, rpt)],
                        degd_hbm.at[pl.ds(base, rpt)])

    f = pl.kernel(
        body,
        out_type=(jax.ShapeDtypeStruct((NC * r_acc,), jnp.float32),
                  jax.ShapeDtypeStruct((NC * r_acc,), jnp.float32)),
        mesh=_mesh(),
        scratch_types=[
            pltpu.VMEM((ch, LN), jnp.int32),
            pltpu.VMEM((ch, LN), jnp.int32),
            pltpu.VMEM((LN,), jnp.float32),
            pltpu.VMEM((-(-rpt // 16) * 16,), jnp.float32),
            pltpu.SemaphoreType.DMA,
            pltpu.VMEM_SHARED((r_acc,), jnp.float32),
            pltpu.VMEM_SHARED((r_acc,), jnp.float32),
        ],
    )
    return f(src_p, dst_p)


def _sc_aggregate(y2, src_p, dst_p):
    return _sc_aggregate_kernel(y2.shape[0] // 2, y2.shape[1],
                                dst_p.shape[1])(y2, src_p, dst_p)


@functools.lru_cache(maxsize=None)
def _sc_aggregate_kernel(r_acc, hd, ch):
    d = 2 * hd
    rpt = r_acc // NS
    NB = 8
    NA = NB // 2

    def body(y_hbm, src_hbm, dst_hbm, out_hbm,
             src_v, dst_v, gbufs, gsem, ssem, acc):
        c = lax.axis_index("c")
        s = lax.axis_index("s")
        pltpu.sync_copy(src_hbm.at[c, s], src_v)
        pltpu.sync_copy(dst_hbm.at[s], dst_v)

        def _z(i, carry):
            for k in range(hd // 16):
                gbufs[0][i, pl.ds(k * 16, 16)] = jnp.zeros((16,), jnp.float32)
            return carry
        lax.fori_loop(0, LN, _z, 0)
        for k in range(rpt // LN):
            pltpu.sync_copy(gbufs[0], acc.at[pl.ds(s * rpt + k * LN, LN)])
        if rpt % LN:
            pltpu.sync_copy(
                gbufs[0].at[pl.ds(0, rpt % LN)],
                acc.at[pl.ds(s * rpt + (rpt // LN) * LN, rpt % LN)])
        plsc.subcore_barrier()

        for j in range(NA):
            pltpu.async_copy(y_hbm.at[src_v.at[j]], gbufs[j], gsem)

        def _round(qq, carry):
            for b in range(NB):
                j = qq * NB + b
                pltpu.make_async_copy(y_hbm.at[src_v.at[j]], gbufs[b],
                                      gsem).wait()
                pltpu.async_copy(gbufs[b], acc.at[dst_v.at[j]], ssem,
                                 add=True)
                nb = (b + NA) % NB

                @pl.when(j + NA < ch)
                def _():
                    @pl.when(j >= NB - NA)
                    def _():
                        pltpu.make_async_copy(
                            gbufs[nb], acc.at[dst_v.at[j - (NB - NA)]],
                            ssem).wait()
                    pltpu.async_copy(y_hbm.at[src_v.at[j + NA]], gbufs[nb],
                                     gsem)
            return carry
        lax.fori_loop(0, ch // NB, _round, 0)
        for k in range(ch - NB, ch):
            pltpu.make_async_copy(gbufs[k % NB], acc.at[dst_v.at[k]],
                                  ssem).wait()

        plsc.subcore_barrier()
        pltpu.sync_copy(acc.at[pl.ds(s * rpt, rpt)],
                        out_hbm.at[pl.ds(s * rpt, rpt), pl.ds(c * hd, hd)])

    f = pl.kernel(
        body,
        out_type=jax.ShapeDtypeStruct((r_acc, d), jnp.float32),
        mesh=_mesh(),
        scratch_types=[
            pltpu.VMEM((ch, LN), jnp.int32),
            pltpu.VMEM((ch, LN), jnp.int32),
            [pltpu.VMEM((LN, hd), jnp.float32) for _ in range(NB)],
            pltpu.SemaphoreType.DMA,
            pltpu.SemaphoreType.DMA,
            pltpu.VMEM_SHARED((r_acc, hd), jnp.float32),
        ],
        compiler_params=pltpu.CompilerParams(use_tc_tiling_on_sc=False),
    )
    return f


def _tc_prescale(x, nsrc):
    def body(x_ref, n_ref, o_ref):
        o_ref[...] = x_ref[...] * n_ref[...]
    return pl.pallas_call(
        body, out_shape=jax.ShapeDtypeStruct(x.shape, jnp.float32),
    )(x, nsrc)


def _tc_mid(m, ndst, nsrc, w, b):
    def body(m_ref, nd_ref, ns_ref, w_ref, b_ref, o_ref):
        h = m_ref[...] * nd_ref[...]
        h = jnp.dot(h, w_ref[...], preferred_element_type=jnp.float32) + b_ref[...]
        o_ref[...] = jnp.maximum(h, 0.0) * ns_ref[...]
    return pl.pallas_call(
        body, out_shape=jax.ShapeDtypeStruct(m.shape, jnp.float32),
    )(m, ndst, nsrc, w, b)


def _tc_final(m, ndst, w, b, n):
    def body(m_ref, nd_ref, w_ref, b_ref, o_ref):
        h = m_ref[:n] * nd_ref[:n]
        o_ref[...] = jnp.dot(h, w_ref[...], preferred_element_type=jnp.float32) + b_ref[...]
    return pl.pallas_call(
        body, out_shape=jax.ShapeDtypeStruct((n, m.shape[1]), jnp.float32),
    )(m, ndst, w, b)


def _gcn(features, edge_index, W1, b1, W2, b2):
    n, d = features.shape
    e = edge_index.shape[1]
    hd = d // 2

    rpt = -(-n // (NS * 8)) * 8
    if rpt * NS == n:
        rpt += 8
    r_acc = rpt * NS
    n_dummy = r_acc - n

    src = edge_index[0]
    dst = edge_index[1]

    r_deg = -(-n // (NS * LN)) * LN * NS
    chd = -(-e // (NW * LN))
    e_padd = NW * chd * LN
    pad_d = n + (jnp.arange(e_padd - e, dtype=jnp.int32) % (r_deg - n))
    src_pd = jnp.concatenate([src, pad_d]).reshape(NW, chd, LN)
    dst_pd = jnp.concatenate([dst, pad_d]).reshape(NW, chd, LN)

    cha = -(-e // (NS * LN))
    cha = (cha + 15) // 16 * 16
    e_pada = NS * cha * LN
    pad_a = n + (jnp.arange(e_pada - e, dtype=jnp.int32) % n_dummy)
    src_a = jnp.concatenate([src, pad_a])
    dst_a = jnp.concatenate([dst, pad_a])
    src_pa = jnp.stack([2 * src_a, 2 * src_a + 1]).reshape(NC, NS, cha, LN)
    dst_pa = dst_a.reshape(NS, cha, LN)

    degs_p, degd_p = _sc_degree(src_pd, dst_pd, r_deg)
    deg_out = (degs_p[:r_deg] + degs_p[r_deg:])[:r_acc]
    deg_in = (degd_p[:r_deg] + degd_p[r_deg:])[:r_acc]
    nsrc = lax.rsqrt(jnp.clip(deg_out, 1.0))[:, None]
    ndst = lax.rsqrt(jnp.clip(deg_in, 1.0))[:, None]

    x_pad = jnp.pad(features, ((0, r_acc - n), (0, 0)))
    y1 = _tc_prescale(x_pad, nsrc)
    m1 = _sc_aggregate(y1.reshape(2 * r_acc, hd), src_pa, dst_pa)
    z = _tc_mid(m1, ndst, nsrc, W1, b1.reshape(1, d))
    m2 = _sc_aggregate(z.reshape(2 * r_acc, hd), src_pa, dst_pa)
    return _tc_final(m2, ndst, W2, b2.reshape(1, d), n)


def kernel(features, edge_index, W1, b1, W2, b2):
    return _gcn(features, edge_index, W1, b1, W2, b2)


# speedup vs baseline: 1.0085x; 1.0085x over previous
import functools

import jax
import jax.numpy as jnp
from jax import lax
from jax.experimental import pallas as pl
from jax.experimental.pallas import tpu as pltpu
from jax.experimental.pallas import tpu_sc as plsc

NC = 2
NS = 16
NW = NC * NS
LN = 128


def _mesh():
    return plsc.VectorSubcoreMesh(
        core_axis_name="c", subcore_axis_name="s", num_cores=NC, num_subcores=NS
    )


def _sc_degree(src_p, dst_p, r_acc):
    ch = src_p.shape[1]
    rpt = r_acc // NS

    def body(src_hbm, dst_hbm, degs_hbm, degd_hbm,
             src_v, dst_v, ones_v, zv, dsem, degs_sh, degd_sh):
        c = lax.axis_index("c")
        s = lax.axis_index("s")
        wid = s * NC + c
        pltpu.sync_copy(src_hbm.at[wid], src_v)
        pltpu.sync_copy(dst_hbm.at[wid], dst_v)
        for k in range(LN // 16):
            ones_v[pl.ds(k * 16, 16)] = jnp.ones((16,), jnp.float32)

        def _z(i, carry):
            zv[pl.ds(i * 16, 16)] = jnp.zeros((16,), jnp.float32)
            return carry
        lax.fori_loop(0, rpt // 16, _z, 0)
        pltpu.sync_copy(zv, degs_sh.at[pl.ds(s * rpt, rpt)])
        pltpu.sync_copy(zv, degd_sh.at[pl.ds(s * rpt, rpt)])
        plsc.subcore_barrier()

        def _chunk(j, carry):
            pltpu.async_copy(ones_v, degs_sh.at[src_v.at[j]], dsem, add=True)
            pltpu.async_copy(ones_v, degd_sh.at[dst_v.at[j]], dsem, add=True)
            return carry
        lax.fori_loop(0, ch, _chunk, 0)

        def _drain(j, carry):
            pltpu.make_async_copy(ones_v, degs_sh.at[src_v.at[j]], dsem).wait()
            pltpu.make_async_copy(ones_v, degd_sh.at[dst_v.at[j]], dsem).wait()
            return carry
        lax.fori_loop(0, ch, _drain, 0)
        plsc.subcore_barrier()
        pltpu.sync_copy(degs_sh.at[pl.ds(s * rpt, rpt)],
                        degs_hbm.at[c, pl.ds(s * rpt, rpt)])
        pltpu.sync_copy(degd_sh.at[pl.ds(s * rpt, rpt)],
                        degd_hbm.at[c, pl.ds(s * rpt, rpt)])

    f = pl.kernel(
        body,
        out_type=(jax.ShapeDtypeStruct((NC, r_acc), jnp.float32),
                  jax.ShapeDtypeStruct((NC, r_acc), jnp.float32)),
        mesh=_mesh(),
        scratch_types=[
            pltpu.VMEM((ch, LN), jnp.int32),
            pltpu.VMEM((ch, LN), jnp.int32),
            pltpu.VMEM((LN,), jnp.float32),
            pltpu.VMEM((rpt,), jnp.float32),
            pltpu.SemaphoreType.DMA,
            pltpu.VMEM_SHARED((r_acc,), jnp.float32),
            pltpu.VMEM_SHARED((r_acc,), jnp.float32),
        ],
    )
    return f(src_p, dst_p)


def _sc_aggregate(y2, src_p, dst_p):
    return _sc_aggregate_kernel(y2.shape[0] // 2, y2.shape[1],
                                dst_p.shape[1])(y2, src_p, dst_p)


@functools.lru_cache(maxsize=None)
def _sc_aggregate_kernel(r_acc, hd, ch):
    d = 2 * hd
    rpt = r_acc // NS
    NB = 8
    NA = NB // 2

    def body(y_hbm, src_hbm, dst_hbm, out_hbm,
             src_v, dst_v, gbufs, gsem, ssem, acc):
        c = lax.axis_index("c")
        s = lax.axis_index("s")
        pltpu.sync_copy(src_hbm.at[c, s], src_v)
        pltpu.sync_copy(dst_hbm.at[s], dst_v)

        def _z(i, carry):
            for k in range(hd // 16):
                gbufs[0][i, pl.ds(k * 16, 16)] = jnp.zeros((16,), jnp.float32)
            return carry
        lax.fori_loop(0, LN, _z, 0)
        for k in range(rpt // LN):
            pltpu.sync_copy(gbufs[0], acc.at[pl.ds(s * rpt + k * LN, LN)])
        plsc.subcore_barrier()

        for j in range(NA):
            pltpu.async_copy(y_hbm.at[src_v.at[j]], gbufs[j], gsem)

        def _round(qq, carry):
            for b in range(NB):
                j = qq * NB + b
                pltpu.make_async_copy(y_hbm.at[src_v.at[j]], gbufs[b],
                                      gsem).wait()
                pltpu.async_copy(gbufs[b], acc.at[dst_v.at[j]], ssem,
                                 add=True)
                nb = (b + NA) % NB

                @pl.when(j + NA < ch)
                def _():
                    @pl.when(j >= NB - NA)
                    def _():
                        pltpu.make_async_copy(
                            gbufs[nb], acc.at[dst_v.at[j - (NB - NA)]],
                            ssem).wait()
                    pltpu.async_copy(y_hbm.at[src_v.at[j + NA]], gbufs[nb],
                                     gsem)
            return carry
        lax.fori_loop(0, ch // NB, _round, 0)
        for k in range(ch - NB, ch):
            pltpu.make_async_copy(gbufs[k % NB], acc.at[dst_v.at[k]],
                                  ssem).wait()

        plsc.subcore_barrier()
        pltpu.sync_copy(acc.at[pl.ds(s * rpt, rpt)],
                        out_hbm.at[pl.ds(s * rpt, rpt), pl.ds(c * hd, hd)])

    f = pl.kernel(
        body,
        out_type=jax.ShapeDtypeStruct((r_acc, d), jnp.float32),
        mesh=_mesh(),
        scratch_types=[
            pltpu.VMEM((ch, LN), jnp.int32),
            pltpu.VMEM((ch, LN), jnp.int32),
            [pltpu.VMEM((LN, hd), jnp.float32) for _ in range(NB)],
            pltpu.SemaphoreType.DMA,
            pltpu.SemaphoreType.DMA,
            pltpu.VMEM_SHARED((r_acc, hd), jnp.float32),
        ],
        compiler_params=pltpu.CompilerParams(use_tc_tiling_on_sc=False),
    )
    return f


def _tc_prescale(x, nsrc):
    def body(x_ref, n_ref, o_ref):
        o_ref[...] = x_ref[...] * n_ref[...]
    return pl.pallas_call(
        body, out_shape=jax.ShapeDtypeStruct(x.shape, jnp.float32),
    )(x, nsrc)


def _tc_mid(m, ndst, nsrc, w, b):
    def body(m_ref, nd_ref, ns_ref, w_ref, b_ref, o_ref):
        h = m_ref[...] * nd_ref[...]
        h = jnp.dot(h, w_ref[...], preferred_element_type=jnp.float32) + b_ref[...]
        o_ref[...] = jnp.maximum(h, 0.0) * ns_ref[...]
    return pl.pallas_call(
        body, out_shape=jax.ShapeDtypeStruct(m.shape, jnp.float32),
    )(m, ndst, nsrc, w, b)


def _tc_final(m, ndst, w, b, n):
    def body(m_ref, nd_ref, w_ref, b_ref, o_ref):
        h = m_ref[:n] * nd_ref[:n]
        o_ref[...] = jnp.dot(h, w_ref[...], preferred_element_type=jnp.float32) + b_ref[...]
    return pl.pallas_call(
        body, out_shape=jax.ShapeDtypeStruct((n, m.shape[1]), jnp.float32),
    )(m, ndst, w, b)


def _gcn(features, edge_index, W1, b1, W2, b2):
    n, d = features.shape
    e = edge_index.shape[1]
    hd = d // 2

    rpt = -(-n // (NS * LN)) * LN
    r_acc = rpt * NS
    n_dummy = r_acc - n

    src = edge_index[0]
    dst = edge_index[1]

    chd = -(-e // (NW * LN))
    e_padd = NW * chd * LN
    pad_d = n + (jnp.arange(e_padd - e, dtype=jnp.int32) % n_dummy)
    src_pd = jnp.concatenate([src, pad_d]).reshape(NW, chd, LN)
    dst_pd = jnp.concatenate([dst, pad_d]).reshape(NW, chd, LN)

    cha = -(-e // (NS * LN))
    cha = (cha + 15) // 16 * 16
    e_pada = NS * cha * LN
    pad_a = n + (jnp.arange(e_pada - e, dtype=jnp.int32) % n_dummy)
    src_a = jnp.concatenate([src, pad_a])
    dst_a = jnp.concatenate([dst, pad_a])
    src_pa = jnp.stack([2 * src_a, 2 * src_a + 1]).reshape(NC, NS, cha, LN)
    dst_pa = dst_a.reshape(NS, cha, LN)

    degs_p, degd_p = _sc_degree(src_pd, dst_pd, r_acc)
    deg_out = degs_p[0] + degs_p[1]
    deg_in = degd_p[0] + degd_p[1]
    nsrc = lax.rsqrt(jnp.clip(deg_out, 1.0))[:, None]
    ndst = lax.rsqrt(jnp.clip(deg_in, 1.0))[:, None]

    x_pad = jnp.pad(features, ((0, r_acc - n), (0, 0)))
    y1 = _tc_prescale(x_pad, nsrc)
    m1 = _sc_aggregate(y1.reshape(2 * r_acc, hd), src_pa, dst_pa)
    z = _tc_mid(m1, ndst, nsrc, W1, b1.reshape(1, d))
    m2 = _sc_aggregate(z.reshape(2 * r_acc, hd), src_pa, dst_pa)
    return _tc_final(m2, ndst, W2, b2.reshape(1, d), n)


def kernel(features, edge_index, W1, b1, W2, b2):
    return _gcn(features, edge_index, W1, b1, W2, b2)


# speedup vs baseline: 1.0285x; 1.0198x over previous
import functools

import jax
import jax.numpy as jnp
from jax import lax
from jax.experimental import pallas as pl
from jax.experimental.pallas import tpu as pltpu
from jax.experimental.pallas import tpu_sc as plsc

NC = 2
NS = 16
NW = NC * NS
LN = 128


def _mesh():
    return plsc.VectorSubcoreMesh(
        core_axis_name="c", subcore_axis_name="s", num_cores=NC, num_subcores=NS
    )


def _sc_degree(src_p, dst_p, r_acc):
    ch = src_p.shape[1]
    rpt = r_acc // NS

    def body(src_hbm, dst_hbm, degs_hbm, degd_hbm,
             src_v, dst_v, ones_v, zv, dsem, degs_sh, degd_sh):
        c = lax.axis_index("c")
        s = lax.axis_index("s")
        wid = s * NC + c
        pltpu.sync_copy(src_hbm.at[wid], src_v)
        pltpu.sync_copy(dst_hbm.at[wid], dst_v)
        for k in range(LN // 16):
            ones_v[pl.ds(k * 16, 16)] = jnp.ones((16,), jnp.float32)

        def _z(i, carry):
            zv[pl.ds(i * 16, 16)] = jnp.zeros((16,), jnp.float32)
            return carry
        lax.fori_loop(0, rpt // 16, _z, 0)
        pltpu.sync_copy(zv, degs_sh.at[pl.ds(s * rpt, rpt)])
        pltpu.sync_copy(zv, degd_sh.at[pl.ds(s * rpt, rpt)])
        plsc.subcore_barrier()

        def _chunk(j, carry):
            pltpu.async_copy(ones_v, degs_sh.at[src_v.at[j]], dsem, add=True)
            pltpu.async_copy(ones_v, degd_sh.at[dst_v.at[j]], dsem, add=True)
            return carry
        lax.fori_loop(0, ch, _chunk, 0)

        def _drain(j, carry):
            pltpu.make_async_copy(ones_v, degs_sh.at[src_v.at[j]], dsem).wait()
            pltpu.make_async_copy(ones_v, degd_sh.at[dst_v.at[j]], dsem).wait()
            return carry
        lax.fori_loop(0, ch, _drain, 0)
        plsc.subcore_barrier()
        pltpu.sync_copy(degs_sh.at[pl.ds(s * rpt, rpt)],
                        degs_hbm.at[c, pl.ds(s * rpt, rpt)])
        pltpu.sync_copy(degd_sh.at[pl.ds(s * rpt, rpt)],
                        degd_hbm.at[c, pl.ds(s * rpt, rpt)])

    f = pl.kernel(
        body,
        out_type=(jax.ShapeDtypeStruct((NC, r_acc), jnp.float32),
                  jax.ShapeDtypeStruct((NC, r_acc), jnp.float32)),
        mesh=_mesh(),
        scratch_types=[
            pltpu.VMEM((ch, LN), jnp.int32),
            pltpu.VMEM((ch, LN), jnp.int32),
            pltpu.VMEM((LN,), jnp.float32),
            pltpu.VMEM((rpt,), jnp.float32),
            pltpu.SemaphoreType.DMA,
            pltpu.VMEM_SHARED((r_acc,), jnp.float32),
            pltpu.VMEM_SHARED((r_acc,), jnp.float32),
        ],
    )
    return f(src_p, dst_p)


def _sc_aggregate(y2, src_p, dst_p):
    return _sc_aggregate_kernel(y2.shape[0] // 2, y2.shape[1],
                                dst_p.shape[1])(y2, src_p, dst_p)


@functools.lru_cache(maxsize=None)
def _sc_aggregate_kernel(r_acc, hd, ch):
    d = 2 * hd
    rpt = r_acc // NS
    NB = 8
    NA = NB // 2

    def body(y_hbm, src_hbm, dst_hbm, out_hbm,
             src_v, dst_v, gbufs, gsem, ssem, acc):
        c = lax.axis_index("c")
        s = lax.axis_index("s")
        pltpu.async_copy(src_hbm.at[c, s], src_v, gsem)
        pltpu.async_copy(dst_hbm.at[s], dst_v, gsem)

        def _z(i, carry):
            for k in range(hd // 16):
                gbufs[0][i, pl.ds(k * 16, 16)] = jnp.zeros((16,), jnp.float32)
            return carry
        lax.fori_loop(0, LN, _z, 0)
        for k in range(rpt // LN):
            pltpu.async_copy(gbufs[0], acc.at[pl.ds(s * rpt + k * LN, LN)],
                             ssem)
        pltpu.make_async_copy(src_hbm.at[c, s], src_v, gsem).wait()
        pltpu.make_async_copy(dst_hbm.at[s], dst_v, gsem).wait()
        for k in range(rpt // LN):
            pltpu.make_async_copy(
                gbufs[0], acc.at[pl.ds(s * rpt + k * LN, LN)], ssem).wait()
        plsc.subcore_barrier()

        for j in range(NA):
            pltpu.async_copy(y_hbm.at[src_v.at[j]], gbufs[j], gsem)

        def _round(qq, carry):
            for b in range(NB):
                j = qq * NB + b
                pltpu.make_async_copy(y_hbm.at[src_v.at[j]], gbufs[b],
                                      gsem).wait()
                pltpu.async_copy(gbufs[b], acc.at[dst_v.at[j]], ssem,
                                 add=True)
                nb = (b + NA) % NB

                @pl.when(j + NA < ch)
                def _():
                    @pl.when(j >= NB - NA)
                    def _():
                        pltpu.make_async_copy(
                            gbufs[nb], acc.at[dst_v.at[j - (NB - NA)]],
                            ssem).wait()
                    pltpu.async_copy(y_hbm.at[src_v.at[j + NA]], gbufs[nb],
                                     gsem)
            return carry
        lax.fori_loop(0, ch // NB, _round, 0)
        for k in range(ch - NB, ch):
            pltpu.make_async_copy(gbufs[k % NB], acc.at[dst_v.at[k]],
                                  ssem).wait()

        plsc.subcore_barrier()
        pltpu.sync_copy(acc.at[pl.ds(s * rpt, rpt)],
                        out_hbm.at[pl.ds(s * rpt, rpt), pl.ds(c * hd, hd)])

    f = pl.kernel(
        body,
        out_type=jax.ShapeDtypeStruct((r_acc, d), jnp.float32),
        mesh=_mesh(),
        scratch_types=[
            pltpu.VMEM((ch, LN), jnp.int32),
            pltpu.VMEM((ch, LN), jnp.int32),
            [pltpu.VMEM((LN, hd), jnp.float32) for _ in range(NB)],
            pltpu.SemaphoreType.DMA,
            pltpu.SemaphoreType.DMA,
            pltpu.VMEM_SHARED((r_acc, hd), jnp.float32),
        ],
        compiler_params=pltpu.CompilerParams(use_tc_tiling_on_sc=False),
    )
    return f


def _tc_prescale(x, nsrc):
    def body(x_ref, n_ref, o_ref):
        o_ref[...] = x_ref[...] * n_ref[...]
    return pl.pallas_call(
        body, out_shape=jax.ShapeDtypeStruct(x.shape, jnp.float32),
    )(x, nsrc)


def _tc_mid(m, ndst, nsrc, w, b):
    def body(m_ref, nd_ref, ns_ref, w_ref, b_ref, o_ref):
        h = m_ref[...] * nd_ref[...]
        h = jnp.dot(h, w_ref[...], preferred_element_type=jnp.float32) + b_ref[...]
        o_ref[...] = jnp.maximum(h, 0.0) * ns_ref[...]
    return pl.pallas_call(
        body, out_shape=jax.ShapeDtypeStruct(m.shape, jnp.float32),
    )(m, ndst, nsrc, w, b)


def _tc_final(m, ndst, w, b, n):
    def body(m_ref, nd_ref, w_ref, b_ref, o_ref):
        h = m_ref[:n] * nd_ref[:n]
        o_ref[...] = jnp.dot(h, w_ref[...], preferred_element_type=jnp.float32) + b_ref[...]
    return pl.pallas_call(
        body, out_shape=jax.ShapeDtypeStruct((n, m.shape[1]), jnp.float32),
    )(m, ndst, w, b)


def _gcn(features, edge_index, W1, b1, W2, b2):
    n, d = features.shape
    e = edge_index.shape[1]
    hd = d // 2

    rpt = -(-n // (NS * LN)) * LN
    r_acc = rpt * NS
    n_dummy = r_acc - n

    src = edge_index[0]
    dst = edge_index[1]

    chd = -(-e // (NW * LN))
    e_padd = NW * chd * LN
    pad_d = n + (jnp.arange(e_padd - e, dtype=jnp.int32) % n_dummy)
    src_pd = jnp.concatenate([src, pad_d]).reshape(NW, chd, LN)
    dst_pd = jnp.concatenate([dst, pad_d]).reshape(NW, chd, LN)

    cha = -(-e // (NS * LN))
    cha = (cha + 15) // 16 * 16
    e_pada = NS * cha * LN
    pad_a = n + (jnp.arange(e_pada - e, dtype=jnp.int32) % n_dummy)
    src_a = jnp.concatenate([src, pad_a])
    dst_a = jnp.concatenate([dst, pad_a])
    src_pa = jnp.stack([2 * src_a, 2 * src_a + 1]).reshape(NC, NS, cha, LN)
    dst_pa = dst_a.reshape(NS, cha, LN)

    degs_p, degd_p = _sc_degree(src_pd, dst_pd, r_acc)
    deg_out = degs_p[0] + degs_p[1]
    deg_in = degd_p[0] + degd_p[1]
    nsrc = lax.rsqrt(jnp.clip(deg_out, 1.0))[:, None]
    ndst = lax.rsqrt(jnp.clip(deg_in, 1.0))[:, None]

    x_pad = jnp.pad(features, ((0, r_acc - n), (0, 0)))
    y1 = _tc_prescale(x_pad, nsrc)
    m1 = _sc_aggregate(y1.reshape(2 * r_acc, hd), src_pa, dst_pa)
    z = _tc_mid(m1, ndst, nsrc, W1, b1.reshape(1, d))
    m2 = _sc_aggregate(z.reshape(2 * r_acc, hd), src_pa, dst_pa)
    return _tc_final(m2, ndst, W2, b2.reshape(1, d), n)


def kernel(features, edge_index, W1, b1, W2, b2):
    return _gcn(features, edge_index, W1, b1, W2, b2)


# speedup vs baseline: 1.0297x; 1.0012x over previous
import functools

import jax
import jax.numpy as jnp
from jax import lax
from jax.experimental import pallas as pl
from jax.experimental.pallas import tpu as pltpu
from jax.experimental.pallas import tpu_sc as plsc

NC = 2
NS = 16
NW = NC * NS
LN = 128


def _mesh():
    return plsc.VectorSubcoreMesh(
        core_axis_name="c", subcore_axis_name="s", num_cores=NC, num_subcores=NS
    )


def _sc_degree(src_p, dst_p, r_acc):
    ch = src_p.shape[1]
    rpt = r_acc // NS

    def body(src_hbm, dst_hbm, degs_hbm, degd_hbm,
             src_v, dst_v, ones_v, zv, dsem, degs_sh, degd_sh):
        c = lax.axis_index("c")
        s = lax.axis_index("s")
        wid = s * NC + c
        pltpu.async_copy(src_hbm.at[wid], src_v, dsem)
        pltpu.async_copy(dst_hbm.at[wid], dst_v, dsem)
        for k in range(LN // 16):
            ones_v[pl.ds(k * 16, 16)] = jnp.ones((16,), jnp.float32)

        def _z(i, carry):
            zv[pl.ds(i * 16, 16)] = jnp.zeros((16,), jnp.float32)
            return carry
        lax.fori_loop(0, rpt // 16, _z, 0)
        pltpu.sync_copy(zv, degs_sh.at[pl.ds(s * rpt, rpt)])
        pltpu.sync_copy(zv, degd_sh.at[pl.ds(s * rpt, rpt)])
        pltpu.make_async_copy(src_hbm.at[wid], src_v, dsem).wait()
        pltpu.make_async_copy(dst_hbm.at[wid], dst_v, dsem).wait()
        plsc.subcore_barrier()

        def _chunk(j, carry):
            pltpu.async_copy(ones_v, degs_sh.at[src_v.at[j]], dsem, add=True)
            pltpu.async_copy(ones_v, degd_sh.at[dst_v.at[j]], dsem, add=True)
            return carry
        lax.fori_loop(0, ch, _chunk, 0)

        def _drain(j, carry):
            pltpu.make_async_copy(ones_v, degs_sh.at[src_v.at[j]], dsem).wait()
            pltpu.make_async_copy(ones_v, degd_sh.at[dst_v.at[j]], dsem).wait()
            return carry
        lax.fori_loop(0, ch, _drain, 0)
        plsc.subcore_barrier()
        pltpu.sync_copy(degs_sh.at[pl.ds(s * rpt, rpt)],
                        degs_hbm.at[c, pl.ds(s * rpt, rpt)])
        pltpu.sync_copy(degd_sh.at[pl.ds(s * rpt, rpt)],
                        degd_hbm.at[c, pl.ds(s * rpt, rpt)])

    f = pl.kernel(
        body,
        out_type=(jax.ShapeDtypeStruct((NC, r_acc), jnp.float32),
                  jax.ShapeDtypeStruct((NC, r_acc), jnp.float32)),
        mesh=_mesh(),
        scratch_types=[
            pltpu.VMEM((ch, LN), jnp.int32),
            pltpu.VMEM((ch, LN), jnp.int32),
            pltpu.VMEM((LN,), jnp.float32),
            pltpu.VMEM((rpt,), jnp.float32),
            pltpu.SemaphoreType.DMA,
            pltpu.VMEM_SHARED((r_acc,), jnp.float32),
            pltpu.VMEM_SHARED((r_acc,), jnp.float32),
        ],
    )
    return f(src_p, dst_p)


def _sc_aggregate(y2, src_p, dst_p):
    return _sc_aggregate_kernel(y2.shape[0] // 2, y2.shape[1],
                                dst_p.shape[1])(y2, src_p, dst_p)


@functools.lru_cache(maxsize=None)
def _sc_aggregate_kernel(r_acc, hd, ch):
    d = 2 * hd
    rpt = r_acc // NS
    NB = 4
    NA = 3

    def body(y_hbm, src_hbm, dst_hbm, out_hbm,
             src_v, dst_v, gbufs, gsem, ssem, acc):
        c = lax.axis_index("c")
        s = lax.axis_index("s")
        pltpu.async_copy(src_hbm.at[c, s], src_v, gsem)
        pltpu.async_copy(dst_hbm.at[s], dst_v, gsem)

        def _z(i, carry):
            for k in range(hd // 16):
                gbufs[0][i, pl.ds(k * 16, 16)] = jnp.zeros((16,), jnp.float32)
            return carry
        lax.fori_loop(0, LN, _z, 0)
        for k in range(rpt // LN):
            pltpu.async_copy(gbufs[0], acc.at[pl.ds(s * rpt + k * LN, LN)],
                             ssem)
        pltpu.make_async_copy(src_hbm.at[c, s], src_v, gsem).wait()
        pltpu.make_async_copy(dst_hbm.at[s], dst_v, gsem).wait()
        for k in range(rpt // LN):
            pltpu.make_async_copy(
                gbufs[0], acc.at[pl.ds(s * rpt + k * LN, LN)], ssem).wait()
        plsc.subcore_barrier()

        for j in range(NA):
            pltpu.async_copy(y_hbm.at[src_v.at[j]], gbufs[j], gsem)

        def _round(qq, carry):
            for b in range(NB):
                j = qq * NB + b
                pltpu.make_async_copy(y_hbm.at[src_v.at[j]], gbufs[b],
                                      gsem).wait()
                pltpu.async_copy(gbufs[b], acc.at[dst_v.at[j]], ssem,
                                 add=True)
                nb = (b + NA) % NB

                @pl.when(j + NA < ch)
                def _():
                    @pl.when(j >= NB - NA)
                    def _():
                        pltpu.make_async_copy(
                            gbufs[nb], acc.at[dst_v.at[j - (NB - NA)]],
                            ssem).wait()
                    pltpu.async_copy(y_hbm.at[src_v.at[j + NA]], gbufs[nb],
                                     gsem)
            return carry
        lax.fori_loop(0, ch // NB, _round, 0)
        for k in range(ch - NB, ch):
            pltpu.make_async_copy(gbufs[k % NB], acc.at[dst_v.at[k]],
                                  ssem).wait()

        plsc.subcore_barrier()
        pltpu.sync_copy(acc.at[pl.ds(s * rpt, rpt)],
                        out_hbm.at[pl.ds(s * rpt, rpt), pl.ds(c * hd, hd)])

    f = pl.kernel(
        body,
        out_type=jax.ShapeDtypeStruct((r_acc, d), jnp.float32),
        mesh=_mesh(),
        scratch_types=[
            pltpu.VMEM((ch, LN), jnp.int32),
            pltpu.VMEM((ch, LN), jnp.int32),
            [pltpu.VMEM((LN, hd), jnp.float32) for _ in range(NB)],
            pltpu.SemaphoreType.DMA,
            pltpu.SemaphoreType.DMA,
            pltpu.VMEM_SHARED((r_acc, hd), jnp.float32),
        ],
        compiler_params=pltpu.CompilerParams(use_tc_tiling_on_sc=False),
    )
    return f


def _tc_prescale(x, nsrc):
    def body(x_ref, n_ref, o_ref):
        o_ref[...] = x_ref[...] * n_ref[...]
    return pl.pallas_call(
        body, out_shape=jax.ShapeDtypeStruct(x.shape, jnp.float32),
    )(x, nsrc)


def _tc_mid(m, ndst, nsrc, w, b):
    def body(m_ref, nd_ref, ns_ref, w_ref, b_ref, o_ref):
        h = m_ref[...] * nd_ref[...]
        h = jnp.dot(h, w_ref[...], preferred_element_type=jnp.float32) + b_ref[...]
        o_ref[...] = jnp.maximum(h, 0.0) * ns_ref[...]
    return pl.pallas_call(
        body, out_shape=jax.ShapeDtypeStruct(m.shape, jnp.float32),
    )(m, ndst, nsrc, w, b)


def _tc_final(m, ndst, w, b, n):
    def body(m_ref, nd_ref, w_ref, b_ref, o_ref):
        h = m_ref[:n] * nd_ref[:n]
        o_ref[...] = jnp.dot(h, w_ref[...], preferred_element_type=jnp.float32) + b_ref[...]
    return pl.pallas_call(
        body, out_shape=jax.ShapeDtypeStruct((n, m.shape[1]), jnp.float32),
    )(m, ndst, w, b)


def _gcn(features, edge_index, W1, b1, W2, b2):
    n, d = features.shape
    e = edge_index.shape[1]
    hd = d // 2

    rpt = -(-n // (NS * LN)) * LN
    r_acc = rpt * NS
    n_dummy = r_acc - n

    src = edge_index[0]
    dst = edge_index[1]

    chd = -(-e // (NW * LN))
    e_padd = NW * chd * LN
    pad_d = n + (jnp.arange(e_padd - e, dtype=jnp.int32) % n_dummy)
    src_pd = jnp.concatenate([src, pad_d]).reshape(NW, chd, LN)
    dst_pd = jnp.concatenate([dst, pad_d]).reshape(NW, chd, LN)

    cha = -(-e // (NS * LN))
    cha = (cha + 15) // 16 * 16
    e_pada = NS * cha * LN
    pad_a = n + (jnp.arange(e_pada - e, dtype=jnp.int32) % n_dummy)
    src_a = jnp.concatenate([src, pad_a])
    dst_a = jnp.concatenate([dst, pad_a])
    src_pa = jnp.stack([2 * src_a, 2 * src_a + 1]).reshape(NC, NS, cha, LN)
    dst_pa = dst_a.reshape(NS, cha, LN)

    degs_p, degd_p = _sc_degree(src_pd, dst_pd, r_acc)
    deg_out = degs_p[0] + degs_p[1]
    deg_in = degd_p[0] + degd_p[1]
    nsrc = lax.rsqrt(jnp.clip(deg_out, 1.0))[:, None]
    ndst = lax.rsqrt(jnp.clip(deg_in, 1.0))[:, None]

    x_pad = jnp.pad(features, ((0, r_acc - n), (0, 0)))
    y1 = _tc_prescale(x_pad, nsrc)
    m1 = _sc_aggregate(y1.reshape(2 * r_acc, hd), src_pa, dst_pa)
    z = _tc_mid(m1, ndst, nsrc, W1, b1.reshape(1, d))
    m2 = _sc_aggregate(z.reshape(2 * r_acc, hd), src_pa, dst_pa)
    return _tc_final(m2, ndst, W2, b2.reshape(1, d), n)


def kernel(features, edge_index, W1, b1, W2, b2):
    return _gcn(features, edge_index, W1, b1, W2, b2)
